# Initial kernel scaffold; baseline (speedup 1.0000x reference)
import jax
import jax.numpy as jnp
from jax.experimental import pallas as pl


def kernel(x_ms, hp_pan, x_pan, g_w, g_b, theta_w, theta_b, phi_w, phi_b, W_w, W_b, bn_w, bn_b):
    raise NotImplementedError("write your pallas kernel here")


# speedup vs baseline: 52.5792x; 52.5792x over previous
import jax
import jax.numpy as jnp
from jax.experimental import pallas as pl
from jax.experimental.pallas import tpu as pltpu


def _conv2d(x, w, b, stride=1, padding=0):
    out = jax.lax.conv_general_dilated(
        x, w, window_strides=(stride, stride),
        padding=[(padding, padding), (padding, padding)],
        dimension_numbers=('NCHW', 'OIHW', 'NCHW'))
    return out + b[None, :, None, None]


def _maxpool2(x):
    return jax.lax.reduce_window(x, -jnp.inf, jax.lax.max,
                                 (1, 1, 2, 2), (1, 1, 2, 2), 'VALID')


def _round_bf16(x):
    u = jax.lax.bitcast_convert_type(x, jnp.int32)
    u = u + 0x7FFF + ((u >> 16) & 1)
    u = jax.lax.bitwise_and(u, jnp.int32(-65536))
    return jax.lax.bitcast_convert_type(u, jnp.float32)


def _attn_core(theta_ref, phi_ref, gsum_ref, gtl_ref, wmat_ref, wb_ref,
               bnw_ref, bnb_ref, z_ref):
    nb = theta_ref.shape[0]
    npix = theta_ref.shape[1]
    wys = []
    total = jnp.zeros((1, 64), dtype=jnp.float32)
    total_sq = jnp.zeros((1, 64), dtype=jnp.float32)
    for b in range(nb):
        f = jnp.dot(theta_ref[b], phi_ref[b],
                    preferred_element_type=jnp.float32)
        m1 = jnp.max(f, axis=1, keepdims=True)
        is1 = f >= m1
        f2 = jnp.where(is1, -jnp.inf, f)
        m2 = jnp.max(f2, axis=1, keepdims=True)
        e2 = jnp.exp(m2 - m1)
        denom = 4.0 + e2
        w1 = _round_bf16(1.0 / denom)
        w2 = _round_bf16(e2 / denom)
        p1 = jnp.where(is1, w1, 0.0)
        p2 = jnp.where(f2 >= m2, w2, 0.0)
        y = (jnp.dot(p1, gsum_ref[b], preferred_element_type=jnp.float32,
                     precision=jax.lax.Precision.HIGHEST)
             + jnp.dot(p2, gtl_ref[b], preferred_element_type=jnp.float32,
                       precision=jax.lax.Precision.HIGHEST))
        wy = jnp.dot(y.astype(jnp.bfloat16),
                     wmat_ref[...].astype(jnp.bfloat16),
                     preferred_element_type=jnp.float32) + wb_ref[...]
        wys.append(wy)
        total = total + jnp.sum(wy, axis=0, keepdims=True)
        total_sq = total_sq + jnp.sum(wy * wy, axis=0, keepdims=True)
    count = float(nb * npix)
    mean = total / count
    var = total_sq / count - mean * mean
    inv = bnw_ref[...] * jax.lax.rsqrt(var + 1e-5)
    shift = bnb_ref[...] - mean * inv
    for b in range(nb):
        z_ref[b] = wys[b] * inv + shift


def kernel(x_ms, hp_pan, x_pan, g_w, g_b, theta_w, theta_b,
           phi_w, phi_b, W_w, W_b, bn_w, bn_b):
    B = x_ms.shape[0]
    C_INTER = g_w.shape[0]

    g_pool = _maxpool2(_conv2d(hp_pan, g_w, g_b))
    theta_s = _conv2d(x_ms, theta_w, theta_b,
                      stride=2, padding=1)
    phi_s = _maxpool2(_conv2d(x_pan, phi_w, phi_b,
                              stride=2, padding=1))

    g_poolb = _round_bf16(g_pool)
    gsum = (g_poolb[:, :, 0::2, 0::2] + g_poolb[:, :, 0::2, 1::2]
            + g_poolb[:, :, 1::2, 0::2] + g_poolb[:, :, 1::2, 1::2])
    gtl = g_poolb[:, :, 0::2, 0::2]

    theta_m = theta_s.reshape(B, C_INTER, -1).transpose(0, 2, 1)
    phi_m = phi_s.reshape(B, C_INTER, -1)
    gsum_m = gsum.reshape(B, C_INTER, -1).transpose(0, 2, 1)
    gtl_m = gtl.reshape(B, C_INTER, -1).transpose(0, 2, 1)

    wmat = W_w.reshape(W_w.shape[0], C_INTER).T
    wb = W_b.reshape(1, -1)
    bnw = bn_w.reshape(1, -1)
    bnb = bn_b.reshape(1, -1)

    z_small = pl.pallas_call(
        _attn_core,
        out_shape=jax.ShapeDtypeStruct((B, 1024, W_w.shape[0]), jnp.float32),
    )(theta_m, phi_m, gsum_m, gtl_m, wmat, wb, bnw, bnb)

    z = z_small.transpose(0, 2, 1).reshape(B, W_w.shape[0], 32, 32)
    z = jnp.repeat(jnp.repeat(z, 2, axis=2), 2, axis=3)
    return (z, x_pan)
